# baseline (speedup 1.0000x reference)
import jax
import jax.numpy as jnp
from jax.experimental import pallas as pl
from jax.experimental.pallas import tpu as pltpu

_G = 50
_A = 20000
_P = 64
_NEG = 64
_HI = jax.lax.Precision.HIGHEST


def _tr(x):
    n = x.shape[0]
    i0 = jax.lax.broadcasted_iota(jnp.int32, (n, n), 0)
    i1 = jax.lax.broadcasted_iota(jnp.int32, (n, n), 1)
    eye = (i0 == i1).astype(x.dtype)
    return jax.lax.dot_general(x, eye, (((0,), (0,)), ((), ())),
                               precision=_HI, preferred_element_type=jnp.float32)


def _dgT(a, b):
    return jax.lax.dot_general(a, b, (((1,), (1,)), ((), ())),
                               precision=_HI, preferred_element_type=jnp.float32)


def _cumsum_lanes(x):
    rows, cols = x.shape
    c = x
    d = 1
    while d < cols:
        z = jnp.zeros((rows, d), dtype=x.dtype)
        c = c + jnp.concatenate([z, c[:, : cols - d]], axis=1)
        d *= 2
    return c


def _body(gtb_ref, gtc_ref, aT_ref, vix_ref,
          d_ref, c_ref, i_ref, sd_ref, si_ref, gn_ref, pn_ref, nn_ref):
    gtb = gtb_ref[0]
    gtc = gtc_ref[0]
    aT = aT_ref[0]
    vix = vix_ref[0].astype(jnp.float32)

    gy1 = gtb[:, 0:1]
    gx1 = gtb[:, 1:2]
    gy2 = gtb[:, 2:3]
    gx2 = gtb[:, 3:4]
    gt_valid = gtb[:, 4:5] > 0.0
    cls = gtc[:, 0:1]
    ay1 = aT[0:1, :]
    ax1 = aT[1:2, :]
    ay2 = aT[2:3, :]
    ax2 = aT[3:4, :]

    iw = jnp.maximum(0.0, jnp.minimum(gx2, ax2) - jnp.maximum(gx1, ax1))
    ih = jnp.maximum(0.0, jnp.minimum(gy2, ay2) - jnp.maximum(gy1, ay1))
    inter = iw * ih
    area_g = (gx2 - gx1) * (gy2 - gy1)
    area_a = (ax2 - ax1) * (ay2 - ay1)
    iou = inter / (area_g + area_a - inter)
    iou = jnp.where(gt_valid, iou, -1.0)

    gt_max = jnp.max(iou, axis=1, keepdims=True)
    a_max = jnp.max(iou, axis=0, keepdims=True)
    a_thr = jnp.where(a_max >= 0.7, a_max, 1.0)
    pos_mat = ((iou == gt_max) | (iou == a_thr)) & gt_valid
    mask = pos_mat.astype(jnp.float32)

    csum = _cumsum_lanes(mask)
    r = csum[:, _A - 1:_A]
    li0 = jax.lax.broadcasted_iota(jnp.int32, (_G, _G), 0)
    li1 = jax.lax.broadcasted_iota(jnp.int32, (_G, _G), 1)
    ltri = (li1 < li0).astype(jnp.float32)
    off = jnp.dot(ltri, r, precision=_HI, preferred_element_type=jnp.float32)
    total_pos = jnp.sum(r)

    offT = _tr(off)
    rT = _tr(r)
    kio = jax.lax.broadcasted_iota(jnp.int32, (_P, _G), 0).astype(jnp.float32)
    contains = ((offT <= kio) & (kio < offT + rT)).astype(jnp.float32)
    kvec = jax.lax.broadcasted_iota(jnp.int32, (_P, 1), 0).astype(jnp.float32)
    t_in_row = kvec - jnp.dot(contains, off, precision=_HI,
                              preferred_element_type=jnp.float32)
    cs_slot = jnp.dot(contains, csum, precision=_HI,
                      preferred_element_type=jnp.float32)
    pai = jnp.sum((cs_slot <= t_in_row).astype(jnp.float32),
                  axis=1, keepdims=True)
    pai = jnp.minimum(pai, float(_A - 1))
    slot_valid = kvec < total_pos

    laneP = jax.lax.broadcasted_iota(jnp.int32, (_P, _A), 1).astype(jnp.float32)
    oh_a = (laneP == pai).astype(jnp.float32)
    pos_an = _dgT(oh_a, aT)
    pos_vx = _dgT(oh_a, vix)
    pos_gt = jnp.dot(contains, gtb, precision=_HI,
                     preferred_element_type=jnp.float32)
    pos_cls = jnp.dot(contains, cls, precision=_HI,
                      preferred_element_type=jnp.float32)

    pa_y1 = pos_an[:, 0:1]
    pa_y2 = pos_an[:, 2:3]
    h = pa_y2 - pa_y1
    gt_h = pos_gt[:, 2:3] - pos_gt[:, 0:1]
    cy = (pa_y2 + pa_y1) * 0.5
    gcy = (pos_gt[:, 2:3] + pos_gt[:, 0:1]) * 0.5
    dy = ((gcy - cy) / h) / 0.1
    dh = jnp.log(gt_h / h) / 0.2
    deltas = jnp.concatenate([dy, dh], axis=1)
    deltas = jnp.where(slot_valid, deltas, 0.0)
    pos_cls = jnp.where(slot_valid, pos_cls, 0.0)
    pos_idx = jnp.where(slot_valid, pos_vx, 0.0)

    any_pos = jnp.max(mask, axis=0, keepdims=True) > 0.0
    negm = ((a_max < 0.5) & jnp.logical_not(any_pos)).astype(jnp.float32)
    cn = _cumsum_lanes(negm)
    total_neg = jnp.sum(negm)
    nidx = jnp.sum((cn <= kvec).astype(jnp.float32), axis=1,
                   keepdims=True)
    nidx = jnp.minimum(nidx, float(_A - 1))
    positive_num = jnp.minimum(total_pos, float(_P))
    negative_num = jnp.minimum(jnp.minimum(total_neg, float(_NEG)),
                               float(_P + _NEG) - positive_num)
    neg_valid = (kvec < total_neg) & (kvec < negative_num)

    tag_pos = slot_valid.astype(jnp.float32)
    tag_neg = neg_valid.astype(jnp.float32)
    zeros2 = jnp.zeros((_NEG, 2), dtype=jnp.float32)
    zeros1 = jnp.zeros((_NEG, 1), dtype=jnp.float32)
    d_top = jnp.concatenate([deltas, tag_pos], axis=1)
    d_bot = jnp.concatenate([zeros2, tag_neg], axis=1)
    d_ref[0] = jnp.concatenate([d_top, d_bot], axis=0)
    c_top = jnp.concatenate([pos_cls, tag_pos], axis=1)
    c_bot = jnp.concatenate([zeros1, tag_neg], axis=1)
    c_ref[0] = jnp.concatenate([c_top, c_bot], axis=0)
    i_col0 = jnp.concatenate([pos_idx, jnp.where(neg_valid, nidx, 0.0)], axis=0)
    i_col1 = jnp.concatenate([tag_pos, -tag_neg], axis=0)
    i_ref[0] = jnp.concatenate([i_col0, i_col1], axis=1).astype(jnp.int32)

    laneG = jax.lax.broadcasted_iota(jnp.int32, (_G, _A), 1).astype(jnp.float32)
    mleft = jnp.where(pos_mat, ax1, 1e10)
    minv = jnp.min(mleft, axis=1, keepdims=True)
    lidx = jnp.min(jnp.where(mleft == minv, laneG, 1e9), axis=1, keepdims=True)
    mright = jnp.where(pos_mat, ax1, -1e10)
    maxv = jnp.max(mright, axis=1, keepdims=True)
    ridx = jnp.min(jnp.where(mright == maxv, laneG, 1e9), axis=1, keepdims=True)
    oh_l = (laneG == lidx).astype(jnp.float32)
    oh_r = (laneG == ridx).astype(jnp.float32)
    la = _dgT(oh_l, aT)
    ra = _dgT(oh_r, aT)
    lvx = _dgT(oh_l, vix)
    rvx = _dgT(oh_r, vix)
    w_l = la[:, 3:4] - la[:, 1:2]
    cx_l = (la[:, 3:4] + la[:, 1:2]) * 0.5
    sl = ((gx1 - cx_l) / w_l) / 0.1
    w_r = ra[:, 3:4] - ra[:, 1:2]
    cx_r = (ra[:, 3:4] + ra[:, 1:2]) * 0.5
    sr = ((gx2 - cx_r) / w_r) / 0.1
    gvf = gt_valid.astype(jnp.float32)
    sd = jnp.where(gt_valid, jnp.concatenate([sl, sr], axis=1), 0.0)
    sd_ref[0] = jnp.concatenate([sd, gvf], axis=1)
    siv = jnp.where(gt_valid, jnp.concatenate([lvx, rvx], axis=1), 0.0)
    si_ref[0] = jnp.concatenate([siv, gvf], axis=1).astype(jnp.int32)

    one11 = jnp.ones((1, 1), dtype=jnp.float32)
    gn_ref[0] = one11 * jnp.sum(gvf)
    pn_ref[0] = one11 * positive_num
    nn_ref[0] = one11 * negative_num


def kernel(gt_boxes, gt_cls_ids, anchors, valid_anchors_indices):
    B = gt_boxes.shape[0]
    aT = jnp.transpose(anchors, (0, 2, 1))
    vix = valid_anchors_indices.reshape(B, 1, _A)

    f32 = jnp.float32
    i32 = jnp.int32
    out_shape = [
        jax.ShapeDtypeStruct((B, _P + _NEG, 3), f32),
        jax.ShapeDtypeStruct((B, _P + _NEG, 2), f32),
        jax.ShapeDtypeStruct((B, _P + _NEG, 2), i32),
        jax.ShapeDtypeStruct((B, _G, 3), f32),
        jax.ShapeDtypeStruct((B, _G, 3), i32),
        jax.ShapeDtypeStruct((B, 1, 1), f32),
        jax.ShapeDtypeStruct((B, 1, 1), f32),
        jax.ShapeDtypeStruct((B, 1, 1), f32),
    ]
    in_specs = [
        pl.BlockSpec((1, _G, 5), lambda b: (b, 0, 0)),
        pl.BlockSpec((1, _G, 2), lambda b: (b, 0, 0)),
        pl.BlockSpec((1, 4, _A), lambda b: (b, 0, 0)),
        pl.BlockSpec((1, 1, _A), lambda b: (b, 0, 0)),
    ]
    out_specs = [
        pl.BlockSpec((1, _P + _NEG, 3), lambda b: (b, 0, 0)),
        pl.BlockSpec((1, _P + _NEG, 2), lambda b: (b, 0, 0)),
        pl.BlockSpec((1, _P + _NEG, 2), lambda b: (b, 0, 0)),
        pl.BlockSpec((1, _G, 3), lambda b: (b, 0, 0)),
        pl.BlockSpec((1, _G, 3), lambda b: (b, 0, 0)),
        pl.BlockSpec((1, 1, 1), lambda b: (b, 0, 0)),
        pl.BlockSpec((1, 1, 1), lambda b: (b, 0, 0)),
        pl.BlockSpec((1, 1, 1), lambda b: (b, 0, 0)),
    ]
    outs = pl.pallas_call(
        _body,
        grid=(B,),
        in_specs=in_specs,
        out_specs=out_specs,
        out_shape=out_shape,
        compiler_params=pltpu.CompilerParams(
            dimension_semantics=("parallel",),
            vmem_limit_bytes=128 * 1024 * 1024,
        ),
    )(gt_boxes, gt_cls_ids, aT, vix)
    d, c, i, sd, si, gn, pn, nn = outs
    return (d, c, i, sd, si,
            gn.reshape(B), pn.reshape(B), nn.reshape(B))


# speedup vs baseline: 1.5266x; 1.5266x over previous
import jax
import jax.numpy as jnp
from jax.experimental import pallas as pl
from jax.experimental.pallas import tpu as pltpu

_G = 50
_A = 20000
_P = 64
_NEG = 64
_HI = jax.lax.Precision.HIGHEST


def _tr(x):
    n = x.shape[0]
    i0 = jax.lax.broadcasted_iota(jnp.int32, (n, n), 0)
    i1 = jax.lax.broadcasted_iota(jnp.int32, (n, n), 1)
    eye = (i0 == i1).astype(x.dtype)
    return jax.lax.dot_general(x, eye, (((0,), (0,)), ((), ())),
                               precision=_HI, preferred_element_type=jnp.float32)


def _dgT(a, b):
    return jax.lax.dot_general(a, b, (((1,), (1,)), ((), ())),
                               precision=_HI, preferred_element_type=jnp.float32)


def _cumsum_lanes(x):
    rows, cols = x.shape
    c = x
    d = 1
    while d < cols:
        z = jnp.zeros((rows, d), dtype=x.dtype)
        c = c + jnp.concatenate([z, c[:, : cols - d]], axis=1)
        d *= 2
    return c


def _body(gtb_ref, gtc_ref, aT_ref, vix_ref,
          d_ref, c_ref, i_ref, sd_ref, si_ref, gn_ref, pn_ref, nn_ref):
    gtb = gtb_ref[0]
    gtc = gtc_ref[0]
    aT = aT_ref[0]
    vix = vix_ref[0].astype(jnp.float32)

    gy1 = gtb[:, 0:1]
    gx1 = gtb[:, 1:2]
    gy2 = gtb[:, 2:3]
    gx2 = gtb[:, 3:4]
    gt_valid = gtb[:, 4:5] > 0.0
    cls = gtc[:, 0:1]
    ay1 = aT[0:1, :]
    ax1 = aT[1:2, :]
    ay2 = aT[2:3, :]
    ax2 = aT[3:4, :]

    iw = jnp.maximum(0.0, jnp.minimum(gx2, ax2) - jnp.maximum(gx1, ax1))
    ih = jnp.maximum(0.0, jnp.minimum(gy2, ay2) - jnp.maximum(gy1, ay1))
    inter = iw * ih
    area_g = (gx2 - gx1) * (gy2 - gy1)
    area_a = (ax2 - ax1) * (ay2 - ay1)
    iou = inter / (area_g + area_a - inter)
    iou = jnp.where(gt_valid, iou, -1.0)

    gt_max = jnp.max(iou, axis=1, keepdims=True)
    a_max = jnp.max(iou, axis=0, keepdims=True)
    a_thr = jnp.where(a_max >= 0.7, a_max, 1.0)
    pos_mat = ((iou == gt_max) | (iou == a_thr)) & gt_valid
    mask = pos_mat.astype(jnp.float32)

    csum = _cumsum_lanes(mask)
    r = jnp.sum(mask, axis=1, keepdims=True)
    li0 = jax.lax.broadcasted_iota(jnp.int32, (_G, _G), 0)
    li1 = jax.lax.broadcasted_iota(jnp.int32, (_G, _G), 1)
    ltri = (li1 < li0).astype(jnp.float32)
    off = jnp.dot(ltri, r, precision=_HI, preferred_element_type=jnp.float32)
    total_pos = jnp.sum(r)

    offT = _tr(off)
    rT = _tr(r)
    kio = jax.lax.broadcasted_iota(jnp.int32, (_P, _G), 0).astype(jnp.float32)
    contains = ((offT <= kio) & (kio < offT + rT)).astype(jnp.float32)
    kvec = jax.lax.broadcasted_iota(jnp.int32, (_P, 1), 0).astype(jnp.float32)
    t_in_row = kvec - jnp.dot(contains, off, precision=_HI,
                              preferred_element_type=jnp.float32)
    cs_slot = jnp.dot(contains.astype(jnp.bfloat16),
                      jnp.minimum(csum, 64.0).astype(jnp.bfloat16),
                      preferred_element_type=jnp.float32)
    pai = jnp.sum((cs_slot <= t_in_row).astype(jnp.float32),
                  axis=1, keepdims=True)
    pai = jnp.minimum(pai, float(_A - 1))
    slot_valid = kvec < total_pos

    laneG = jax.lax.broadcasted_iota(jnp.int32, (_G, _A), 1).astype(jnp.float32)
    mleft = jnp.where(pos_mat, ax1, 1e10)
    minv = jnp.min(mleft, axis=1, keepdims=True)
    lidx = jnp.min(jnp.where(mleft == minv, laneG, 1e9), axis=1, keepdims=True)
    mright = jnp.where(pos_mat, ax1, -1e10)
    maxv = jnp.max(mright, axis=1, keepdims=True)
    ridx = jnp.min(jnp.where(mright == maxv, laneG, 1e9), axis=1, keepdims=True)

    laneP = jax.lax.broadcasted_iota(jnp.int32, (_P, _A), 1).astype(jnp.float32)
    oh_a = (laneP == pai).astype(jnp.float32)
    oh_l = (laneG == lidx).astype(jnp.float32)
    oh_r = (laneG == ridx).astype(jnp.float32)
    ohs = jnp.concatenate([oh_a, oh_l, oh_r], axis=0)
    rhs5 = jnp.concatenate([aT, vix], axis=0)
    gat = _dgT(ohs, rhs5)
    pos_an = gat[0:_P, 0:4]
    pos_vx = gat[0:_P, 4:5]
    la = gat[_P:_P + _G, 0:4]
    lvx = gat[_P:_P + _G, 4:5]
    ra = gat[_P + _G:_P + 2 * _G, 0:4]
    rvx = gat[_P + _G:_P + 2 * _G, 4:5]
    pos_gt = jnp.dot(contains, gtb, precision=_HI,
                     preferred_element_type=jnp.float32)
    pos_cls = jnp.dot(contains, cls, precision=_HI,
                      preferred_element_type=jnp.float32)

    pa_y1 = pos_an[:, 0:1]
    pa_y2 = pos_an[:, 2:3]
    h = pa_y2 - pa_y1
    gt_h = pos_gt[:, 2:3] - pos_gt[:, 0:1]
    cy = (pa_y2 + pa_y1) * 0.5
    gcy = (pos_gt[:, 2:3] + pos_gt[:, 0:1]) * 0.5
    dy = ((gcy - cy) / h) / 0.1
    dh = jnp.log(gt_h / h) / 0.2
    deltas = jnp.concatenate([dy, dh], axis=1)
    deltas = jnp.where(slot_valid, deltas, 0.0)
    pos_cls = jnp.where(slot_valid, pos_cls, 0.0)
    pos_idx = jnp.where(slot_valid, pos_vx, 0.0)

    any_pos = jnp.max(mask, axis=0, keepdims=True) > 0.0
    negm = ((a_max < 0.5) & jnp.logical_not(any_pos)).astype(jnp.float32)
    cn = _cumsum_lanes(negm)
    total_neg = jnp.sum(negm)
    nidx = jnp.sum((cn <= kvec).astype(jnp.float32), axis=1,
                   keepdims=True)
    nidx = jnp.minimum(nidx, float(_A - 1))
    positive_num = jnp.minimum(total_pos, float(_P))
    negative_num = jnp.minimum(jnp.minimum(total_neg, float(_NEG)),
                               float(_P + _NEG) - positive_num)
    neg_valid = (kvec < total_neg) & (kvec < negative_num)

    tag_pos = slot_valid.astype(jnp.float32)
    tag_neg = neg_valid.astype(jnp.float32)
    zeros2 = jnp.zeros((_NEG, 2), dtype=jnp.float32)
    zeros1 = jnp.zeros((_NEG, 1), dtype=jnp.float32)
    d_top = jnp.concatenate([deltas, tag_pos], axis=1)
    d_bot = jnp.concatenate([zeros2, tag_neg], axis=1)
    d_ref[0] = jnp.concatenate([d_top, d_bot], axis=0)
    c_top = jnp.concatenate([pos_cls, tag_pos], axis=1)
    c_bot = jnp.concatenate([zeros1, tag_neg], axis=1)
    c_ref[0] = jnp.concatenate([c_top, c_bot], axis=0)
    i_col0 = jnp.concatenate([pos_idx, jnp.where(neg_valid, nidx, 0.0)], axis=0)
    i_col1 = jnp.concatenate([tag_pos, -tag_neg], axis=0)
    i_ref[0] = jnp.concatenate([i_col0, i_col1], axis=1).astype(jnp.int32)

    w_l = la[:, 3:4] - la[:, 1:2]
    cx_l = (la[:, 3:4] + la[:, 1:2]) * 0.5
    sl = ((gx1 - cx_l) / w_l) / 0.1
    w_r = ra[:, 3:4] - ra[:, 1:2]
    cx_r = (ra[:, 3:4] + ra[:, 1:2]) * 0.5
    sr = ((gx2 - cx_r) / w_r) / 0.1
    gvf = gt_valid.astype(jnp.float32)
    sd = jnp.where(gt_valid, jnp.concatenate([sl, sr], axis=1), 0.0)
    sd_ref[0] = jnp.concatenate([sd, gvf], axis=1)
    siv = jnp.where(gt_valid, jnp.concatenate([lvx, rvx], axis=1), 0.0)
    si_ref[0] = jnp.concatenate([siv, gvf], axis=1).astype(jnp.int32)

    one11 = jnp.ones((1, 1), dtype=jnp.float32)
    gn_ref[0] = one11 * jnp.sum(gvf)
    pn_ref[0] = one11 * positive_num
    nn_ref[0] = one11 * negative_num


def kernel(gt_boxes, gt_cls_ids, anchors, valid_anchors_indices):
    B = gt_boxes.shape[0]
    aT = jnp.transpose(anchors, (0, 2, 1))
    vix = valid_anchors_indices.reshape(B, 1, _A)

    f32 = jnp.float32
    i32 = jnp.int32
    out_shape = [
        jax.ShapeDtypeStruct((B, _P + _NEG, 3), f32),
        jax.ShapeDtypeStruct((B, _P + _NEG, 2), f32),
        jax.ShapeDtypeStruct((B, _P + _NEG, 2), i32),
        jax.ShapeDtypeStruct((B, _G, 3), f32),
        jax.ShapeDtypeStruct((B, _G, 3), i32),
        jax.ShapeDtypeStruct((B, 1, 1), f32),
        jax.ShapeDtypeStruct((B, 1, 1), f32),
        jax.ShapeDtypeStruct((B, 1, 1), f32),
    ]
    in_specs = [
        pl.BlockSpec((1, _G, 5), lambda b: (b, 0, 0)),
        pl.BlockSpec((1, _G, 2), lambda b: (b, 0, 0)),
        pl.BlockSpec((1, 4, _A), lambda b: (b, 0, 0)),
        pl.BlockSpec((1, 1, _A), lambda b: (b, 0, 0)),
    ]
    out_specs = [
        pl.BlockSpec((1, _P + _NEG, 3), lambda b: (b, 0, 0)),
        pl.BlockSpec((1, _P + _NEG, 2), lambda b: (b, 0, 0)),
        pl.BlockSpec((1, _P + _NEG, 2), lambda b: (b, 0, 0)),
        pl.BlockSpec((1, _G, 3), lambda b: (b, 0, 0)),
        pl.BlockSpec((1, _G, 3), lambda b: (b, 0, 0)),
        pl.BlockSpec((1, 1, 1), lambda b: (b, 0, 0)),
        pl.BlockSpec((1, 1, 1), lambda b: (b, 0, 0)),
        pl.BlockSpec((1, 1, 1), lambda b: (b, 0, 0)),
    ]
    outs = pl.pallas_call(
        _body,
        grid=(B,),
        in_specs=in_specs,
        out_specs=out_specs,
        out_shape=out_shape,
        compiler_params=pltpu.CompilerParams(
            dimension_semantics=("parallel",),
            vmem_limit_bytes=128 * 1024 * 1024,
        ),
    )(gt_boxes, gt_cls_ids, aT, vix)
    d, c, i, sd, si, gn, pn, nn = outs
    return (d, c, i, sd, si,
            gn.reshape(B), pn.reshape(B), nn.reshape(B))


# speedup vs baseline: 1.9717x; 1.2915x over previous
import jax
import jax.numpy as jnp
from jax.experimental import pallas as pl
from jax.experimental.pallas import tpu as pltpu

_G = 50
_A = 20000
_P = 64
_NEG = 64
_HI = jax.lax.Precision.HIGHEST


def _tr(x):
    n = x.shape[0]
    i0 = jax.lax.broadcasted_iota(jnp.int32, (n, n), 0)
    i1 = jax.lax.broadcasted_iota(jnp.int32, (n, n), 1)
    eye = (i0 == i1).astype(x.dtype)
    return jax.lax.dot_general(x, eye, (((0,), (0,)), ((), ())),
                               precision=_HI, preferred_element_type=jnp.float32)


def _dgT(a, b):
    return jax.lax.dot_general(a, b, (((1,), (1,)), ((), ())),
                               precision=_HI, preferred_element_type=jnp.float32)


def _cumsum_sat64(x):
    rows, cols = x.shape
    cap = jnp.asarray(64.0, dtype=jnp.bfloat16)
    c = x.astype(jnp.bfloat16)
    d = 1
    while d < cols:
        z = jnp.zeros((rows, d), dtype=jnp.bfloat16)
        c = jnp.minimum(c + jnp.concatenate([z, c[:, : cols - d]], axis=1), cap)
        d *= 2
    return c


def _bf16_split3(x):
    r0 = x.astype(jnp.bfloat16)
    rem1 = x - r0.astype(jnp.float32)
    r1 = rem1.astype(jnp.bfloat16)
    rem2 = rem1 - r1.astype(jnp.float32)
    r2 = rem2.astype(jnp.bfloat16)
    return r0, r1, r2


def _dg_bf(a, b):
    return jax.lax.dot_general(a, b, (((1,), (1,)), ((), ())),
                               preferred_element_type=jnp.float32)


def _body(gtb_ref, gtc_ref, aT_ref, vix_ref,
          d_ref, c_ref, i_ref, sd_ref, si_ref, gn_ref, pn_ref, nn_ref):
    gtb = gtb_ref[0]
    gtc = gtc_ref[0]
    aT = aT_ref[0]
    vix = vix_ref[0].astype(jnp.float32)

    gy1 = gtb[:, 0:1]
    gx1 = gtb[:, 1:2]
    gy2 = gtb[:, 2:3]
    gx2 = gtb[:, 3:4]
    gt_valid = gtb[:, 4:5] > 0.0
    cls = gtc[:, 0:1]
    ay1 = aT[0:1, :]
    ax1 = aT[1:2, :]
    ay2 = aT[2:3, :]
    ax2 = aT[3:4, :]

    iw = jnp.maximum(0.0, jnp.minimum(gx2, ax2) - jnp.maximum(gx1, ax1))
    ih = jnp.maximum(0.0, jnp.minimum(gy2, ay2) - jnp.maximum(gy1, ay1))
    inter = iw * ih
    area_g = (gx2 - gx1) * (gy2 - gy1)
    area_a = (ax2 - ax1) * (ay2 - ay1)
    iou = inter / (area_g + area_a - inter)
    iou = jnp.where(gt_valid, iou, -1.0)

    gt_max = jnp.max(iou, axis=1, keepdims=True)
    a_max = jnp.max(iou, axis=0, keepdims=True)
    a_thr = jnp.where(a_max >= 0.7, a_max, 1.0)
    pos_mat = ((iou == gt_max) | (iou == a_thr)) & gt_valid
    mask = pos_mat.astype(jnp.float32)

    csat = _cumsum_sat64(mask)
    r = jnp.sum(mask, axis=1, keepdims=True)
    li0 = jax.lax.broadcasted_iota(jnp.int32, (_G, _G), 0)
    li1 = jax.lax.broadcasted_iota(jnp.int32, (_G, _G), 1)
    ltri = (li1 < li0).astype(jnp.float32)
    off = jnp.dot(ltri, r, precision=_HI, preferred_element_type=jnp.float32)
    total_pos = jnp.sum(r)

    offT = _tr(off)
    rT = _tr(r)
    kio = jax.lax.broadcasted_iota(jnp.int32, (_P, _G), 0).astype(jnp.float32)
    contains = ((offT <= kio) & (kio < offT + rT)).astype(jnp.float32)
    kvec = jax.lax.broadcasted_iota(jnp.int32, (_P, 1), 0).astype(jnp.float32)
    t_in_row = kvec - jnp.dot(contains, off, precision=_HI,
                              preferred_element_type=jnp.float32)
    cs_slot = jnp.dot(contains.astype(jnp.bfloat16), csat,
                      preferred_element_type=jnp.float32)
    pai = jnp.sum((cs_slot <= t_in_row).astype(jnp.float32),
                  axis=1, keepdims=True)
    pai = jnp.minimum(pai, float(_A - 1))
    slot_valid = kvec < total_pos

    laneG = jax.lax.broadcasted_iota(jnp.int32, (_G, _A), 1).astype(jnp.float32)
    mleft = jnp.where(pos_mat, ax1, 1e10)
    minv = jnp.min(mleft, axis=1, keepdims=True)
    lidx = jnp.min(jnp.where(mleft == minv, laneG, 1e9), axis=1, keepdims=True)
    mright = jnp.where(pos_mat, ax1, -1e10)
    maxv = jnp.max(mright, axis=1, keepdims=True)
    ridx = jnp.min(jnp.where(mright == maxv, laneG, 1e9), axis=1, keepdims=True)

    laneP = jax.lax.broadcasted_iota(jnp.int32, (_P, _A), 1).astype(jnp.float32)
    oh_a = (laneP == pai).astype(jnp.float32)
    oh_l = (laneG == lidx).astype(jnp.float32)
    oh_r = (laneG == ridx).astype(jnp.float32)
    ohs = jnp.concatenate([oh_a, oh_l, oh_r], axis=0).astype(jnp.bfloat16)
    rhs5 = jnp.concatenate([aT, vix], axis=0)
    q0, q1, q2 = _bf16_split3(rhs5)
    gat = (_dg_bf(ohs, q0) + _dg_bf(ohs, q1)) + _dg_bf(ohs, q2)
    pos_an = gat[0:_P, 0:4]
    pos_vx = gat[0:_P, 4:5]
    la = gat[_P:_P + _G, 0:4]
    lvx = gat[_P:_P + _G, 4:5]
    ra = gat[_P + _G:_P + 2 * _G, 0:4]
    rvx = gat[_P + _G:_P + 2 * _G, 4:5]
    pos_gt = jnp.dot(contains, gtb, precision=_HI,
                     preferred_element_type=jnp.float32)
    pos_cls = jnp.dot(contains, cls, precision=_HI,
                      preferred_element_type=jnp.float32)

    pa_y1 = pos_an[:, 0:1]
    pa_y2 = pos_an[:, 2:3]
    h = pa_y2 - pa_y1
    gt_h = pos_gt[:, 2:3] - pos_gt[:, 0:1]
    cy = (pa_y2 + pa_y1) * 0.5
    gcy = (pos_gt[:, 2:3] + pos_gt[:, 0:1]) * 0.5
    dy = ((gcy - cy) / h) / 0.1
    dh = jnp.log(gt_h / h) / 0.2
    deltas = jnp.concatenate([dy, dh], axis=1)
    deltas = jnp.where(slot_valid, deltas, 0.0)
    pos_cls = jnp.where(slot_valid, pos_cls, 0.0)
    pos_idx = jnp.where(slot_valid, pos_vx, 0.0)

    any_pos = jnp.max(mask, axis=0, keepdims=True) > 0.0
    negm = ((a_max < 0.5) & jnp.logical_not(any_pos)).astype(jnp.float32)
    cn = _cumsum_sat64(negm).astype(jnp.float32)
    total_neg = jnp.sum(negm)
    nidx = jnp.sum((cn <= kvec).astype(jnp.float32), axis=1,
                   keepdims=True)
    nidx = jnp.minimum(nidx, float(_A - 1))
    positive_num = jnp.minimum(total_pos, float(_P))
    negative_num = jnp.minimum(jnp.minimum(total_neg, float(_NEG)),
                               float(_P + _NEG) - positive_num)
    neg_valid = (kvec < total_neg) & (kvec < negative_num)

    tag_pos = slot_valid.astype(jnp.float32)
    tag_neg = neg_valid.astype(jnp.float32)
    zeros2 = jnp.zeros((_NEG, 2), dtype=jnp.float32)
    zeros1 = jnp.zeros((_NEG, 1), dtype=jnp.float32)
    d_top = jnp.concatenate([deltas, tag_pos], axis=1)
    d_bot = jnp.concatenate([zeros2, tag_neg], axis=1)
    d_ref[0] = jnp.concatenate([d_top, d_bot], axis=0)
    c_top = jnp.concatenate([pos_cls, tag_pos], axis=1)
    c_bot = jnp.concatenate([zeros1, tag_neg], axis=1)
    c_ref[0] = jnp.concatenate([c_top, c_bot], axis=0)
    i_col0 = jnp.concatenate([pos_idx, jnp.where(neg_valid, nidx, 0.0)], axis=0)
    i_col1 = jnp.concatenate([tag_pos, -tag_neg], axis=0)
    i_ref[0] = jnp.concatenate([i_col0, i_col1], axis=1).astype(jnp.int32)

    w_l = la[:, 3:4] - la[:, 1:2]
    cx_l = (la[:, 3:4] + la[:, 1:2]) * 0.5
    sl = ((gx1 - cx_l) / w_l) / 0.1
    w_r = ra[:, 3:4] - ra[:, 1:2]
    cx_r = (ra[:, 3:4] + ra[:, 1:2]) * 0.5
    sr = ((gx2 - cx_r) / w_r) / 0.1
    gvf = gt_valid.astype(jnp.float32)
    sd = jnp.where(gt_valid, jnp.concatenate([sl, sr], axis=1), 0.0)
    sd_ref[0] = jnp.concatenate([sd, gvf], axis=1)
    siv = jnp.where(gt_valid, jnp.concatenate([lvx, rvx], axis=1), 0.0)
    si_ref[0] = jnp.concatenate([siv, gvf], axis=1).astype(jnp.int32)

    one11 = jnp.ones((1, 1), dtype=jnp.float32)
    gn_ref[0] = one11 * jnp.sum(gvf)
    pn_ref[0] = one11 * positive_num
    nn_ref[0] = one11 * negative_num


def kernel(gt_boxes, gt_cls_ids, anchors, valid_anchors_indices):
    B = gt_boxes.shape[0]
    aT = jnp.transpose(anchors, (0, 2, 1))
    vix = valid_anchors_indices.reshape(B, 1, _A)

    f32 = jnp.float32
    i32 = jnp.int32
    out_shape = [
        jax.ShapeDtypeStruct((B, _P + _NEG, 3), f32),
        jax.ShapeDtypeStruct((B, _P + _NEG, 2), f32),
        jax.ShapeDtypeStruct((B, _P + _NEG, 2), i32),
        jax.ShapeDtypeStruct((B, _G, 3), f32),
        jax.ShapeDtypeStruct((B, _G, 3), i32),
        jax.ShapeDtypeStruct((B, 1, 1), f32),
        jax.ShapeDtypeStruct((B, 1, 1), f32),
        jax.ShapeDtypeStruct((B, 1, 1), f32),
    ]
    in_specs = [
        pl.BlockSpec((1, _G, 5), lambda b: (b, 0, 0)),
        pl.BlockSpec((1, _G, 2), lambda b: (b, 0, 0)),
        pl.BlockSpec((1, 4, _A), lambda b: (b, 0, 0)),
        pl.BlockSpec((1, 1, _A), lambda b: (b, 0, 0)),
    ]
    out_specs = [
        pl.BlockSpec((1, _P + _NEG, 3), lambda b: (b, 0, 0)),
        pl.BlockSpec((1, _P + _NEG, 2), lambda b: (b, 0, 0)),
        pl.BlockSpec((1, _P + _NEG, 2), lambda b: (b, 0, 0)),
        pl.BlockSpec((1, _G, 3), lambda b: (b, 0, 0)),
        pl.BlockSpec((1, _G, 3), lambda b: (b, 0, 0)),
        pl.BlockSpec((1, 1, 1), lambda b: (b, 0, 0)),
        pl.BlockSpec((1, 1, 1), lambda b: (b, 0, 0)),
        pl.BlockSpec((1, 1, 1), lambda b: (b, 0, 0)),
    ]
    outs = pl.pallas_call(
        _body,
        grid=(B,),
        in_specs=in_specs,
        out_specs=out_specs,
        out_shape=out_shape,
        compiler_params=pltpu.CompilerParams(
            dimension_semantics=("parallel",),
            vmem_limit_bytes=128 * 1024 * 1024,
        ),
    )(gt_boxes, gt_cls_ids, aT, vix)
    d, c, i, sd, si, gn, pn, nn = outs
    return (d, c, i, sd, si,
            gn.reshape(B), pn.reshape(B), nn.reshape(B))
